# baseline (speedup 1.0000x reference)
import jax
import jax.numpy as jnp
from jax.experimental import pallas as pl
from jax.experimental.pallas import tpu as pltpu

Q = 256
D = 768
N_KEYS = 100000
TOPK = 5
N_BLK = 4000
N_BLOCKS = N_KEYS // N_BLK
SLICE = 8

NEG_INF = float("-inf")
BIG = 2**30


def _retrieve_kernel(q_ref, qn_ref, kb_ref, vals_out, idx_out,
                     raw_ref, kn_ref, rv_ref, ri_ref):
    pid = pl.program_id(0)

    @pl.when(pid == 0)
    def _init():
        rv_ref[...] = jnp.full((TOPK, Q), NEG_INF, jnp.float32)
        ri_ref[...] = jnp.zeros((TOPK, Q), jnp.int32)

    kb = kb_ref[...]
    raw_ref[...] = jax.lax.dot_general(
        kb, q_ref[...],
        dimension_numbers=(((1,), (1,)), ((), ())),
        preferred_element_type=jnp.float32,
    )
    kn_ref[...] = jnp.sqrt(jnp.sum(kb * kb, axis=1, keepdims=True))

    qn_row = qn_ref[0:1]
    base = pid * N_BLK
    iota8 = jax.lax.broadcasted_iota(jnp.int32, (SLICE, Q), 0)

    def body(r, carry):
        s = r * SLICE
        v = raw_ref[pl.ds(s, SLICE), :] / (kn_ref[pl.ds(s, SLICE), :] * qn_row)
        iv = iota8 + (base + s)
        new = []
        for t, ti in carry:
            up = v > t
            nt = jnp.where(up, v, t)
            ni = jnp.where(up, iv, ti)
            v = jnp.where(up, t, v)
            iv = jnp.where(up, ti, iv)
            new.append((nt, ni))
        return tuple(new)

    init = tuple(
        (jnp.full((SLICE, Q), NEG_INF, jnp.float32),
         jnp.full((SLICE, Q), BIG, jnp.int32))
        for _ in range(TOPK))
    tops = jax.lax.fori_loop(0, N_BLK // SLICE, body, init)

    cat_v = jnp.concatenate([rv_ref[...]] + [t for t, _ in tops], axis=0)
    cat_i = jnp.concatenate([ri_ref[...]] + [ti for _, ti in tops], axis=0)
    big = jnp.int32(BIG)
    mv = []
    mi = []
    for _ in range(TOPK):
        m = jnp.max(cat_v, axis=0, keepdims=True)
        a = jnp.min(jnp.where(cat_v == m, cat_i, big), axis=0, keepdims=True)
        mv.append(m)
        mi.append(a)
        cat_v = jnp.where(cat_i == a, NEG_INF, cat_v)
    rv_ref[...] = jnp.concatenate(mv, axis=0)
    ri_ref[...] = jnp.concatenate(mi, axis=0)

    @pl.when(pid == N_BLOCKS - 1)
    def _finish():
        vals_out[...] = rv_ref[...]
        idx_out[...] = ri_ref[...]


@jax.jit
def _retrieve(queries, keys, k):
    qn = jnp.broadcast_to(
        jnp.linalg.norm(queries, axis=1)[None, :], (8, Q))
    vals_t, idx_t = pl.pallas_call(
        _retrieve_kernel,
        grid=(N_BLOCKS,),
        in_specs=[
            pl.BlockSpec((Q, D), lambda i: (0, 0)),
            pl.BlockSpec((8, Q), lambda i: (0, 0)),
            pl.BlockSpec((N_BLK, D), lambda i: (i, 0)),
        ],
        out_specs=[
            pl.BlockSpec((TOPK, Q), lambda i: (0, 0)),
            pl.BlockSpec((TOPK, Q), lambda i: (0, 0)),
        ],
        out_shape=[
            jax.ShapeDtypeStruct((TOPK, Q), jnp.float32),
            jax.ShapeDtypeStruct((TOPK, Q), jnp.int32),
        ],
        scratch_shapes=[
            pltpu.VMEM((N_BLK, Q), jnp.float32),
            pltpu.VMEM((N_BLK, 1), jnp.float32),
            pltpu.VMEM((TOPK, Q), jnp.float32),
            pltpu.VMEM((TOPK, Q), jnp.int32),
        ],
    )(queries, qn, keys)
    return vals_t.T, idx_t.T + (k - TOPK)


def kernel(queries, keys, k):
    return _retrieve(queries, keys, k)


# speedup vs baseline: 4.3906x; 4.3906x over previous
import jax
import jax.numpy as jnp
from jax.experimental import pallas as pl
from jax.experimental.pallas import tpu as pltpu

Q = 256
D = 768
N_KEYS = 100000
TOPK = 5
N_BLK = 4000
N_BLOCKS = N_KEYS // N_BLK
SLICE = 8
CHUNK = 80

NEG_INF = float("-inf")
BIG = 2**30


def _retrieve_kernel(q_ref, qn_ref, kb_ref, vals_out, idx_out,
                     raw_ref, kn_ref, rv_ref, ri_ref):
    pid = pl.program_id(0)

    @pl.when(pid == 0)
    def _init():
        rv_ref[...] = jnp.full((TOPK, Q), NEG_INF, jnp.float32)
        ri_ref[...] = jnp.zeros((TOPK, Q), jnp.int32)

    kb = kb_ref[...]
    raw_ref[...] = jax.lax.dot_general(
        kb, q_ref[...],
        dimension_numbers=(((1,), (1,)), ((), ())),
        preferred_element_type=jnp.float32,
    )
    kn_ref[...] = jnp.sqrt(jnp.sum(kb * kb, axis=1, keepdims=True))

    qn_row = qn_ref[0:1]
    base = pid * N_BLK
    iotac = jax.lax.broadcasted_iota(jnp.int32, (CHUNK, Q), 0)

    def body(r, carry):
        s = r * CHUNK
        vc = raw_ref[pl.ds(s, CHUNK), :] / (kn_ref[pl.ds(s, CHUNK), :] * qn_row)
        ivc = iotac + (base + s)
        carry = list(carry)
        for u in range(CHUNK // SLICE):
            v = vc[u * SLICE:(u + 1) * SLICE]
            iv = ivc[u * SLICE:(u + 1) * SLICE]
            for lvl in range(TOPK):
                t, ti = carry[lvl]
                up = v > t
                nt = jnp.where(up, v, t)
                ni = jnp.where(up, iv, ti)
                v = jnp.where(up, t, v)
                iv = jnp.where(up, ti, iv)
                carry[lvl] = (nt, ni)
        return tuple(carry)

    init = tuple(
        (jnp.full((SLICE, Q), NEG_INF, jnp.float32),
         jnp.full((SLICE, Q), BIG, jnp.int32))
        for _ in range(TOPK))
    tops = jax.lax.fori_loop(0, N_BLK // CHUNK, body, init)

    cat_v = jnp.concatenate([rv_ref[...]] + [t for t, _ in tops], axis=0)
    cat_i = jnp.concatenate([ri_ref[...]] + [ti for _, ti in tops], axis=0)
    big = jnp.int32(BIG)
    mv = []
    mi = []
    for _ in range(TOPK):
        m = jnp.max(cat_v, axis=0, keepdims=True)
        a = jnp.min(jnp.where(cat_v == m, cat_i, big), axis=0, keepdims=True)
        mv.append(m)
        mi.append(a)
        cat_v = jnp.where(cat_i == a, NEG_INF, cat_v)
    rv_ref[...] = jnp.concatenate(mv, axis=0)
    ri_ref[...] = jnp.concatenate(mi, axis=0)

    @pl.when(pid == N_BLOCKS - 1)
    def _finish():
        vals_out[...] = rv_ref[...]
        idx_out[...] = ri_ref[...]


@jax.jit
def _retrieve(queries, keys, k):
    qn = jnp.broadcast_to(
        jnp.linalg.norm(queries, axis=1)[None, :], (8, Q))
    vals_t, idx_t = pl.pallas_call(
        _retrieve_kernel,
        grid=(N_BLOCKS,),
        in_specs=[
            pl.BlockSpec((Q, D), lambda i: (0, 0)),
            pl.BlockSpec((8, Q), lambda i: (0, 0)),
            pl.BlockSpec((N_BLK, D), lambda i: (i, 0)),
        ],
        out_specs=[
            pl.BlockSpec((TOPK, Q), lambda i: (0, 0)),
            pl.BlockSpec((TOPK, Q), lambda i: (0, 0)),
        ],
        out_shape=[
            jax.ShapeDtypeStruct((TOPK, Q), jnp.float32),
            jax.ShapeDtypeStruct((TOPK, Q), jnp.int32),
        ],
        scratch_shapes=[
            pltpu.VMEM((N_BLK, Q), jnp.float32),
            pltpu.VMEM((N_BLK, 1), jnp.float32),
            pltpu.VMEM((TOPK, Q), jnp.float32),
            pltpu.VMEM((TOPK, Q), jnp.int32),
        ],
    )(queries, qn, keys)
    return vals_t.T, idx_t.T + (k - TOPK)


def kernel(queries, keys, k):
    return _retrieve(queries, keys, k)


# speedup vs baseline: 5.0332x; 1.1464x over previous
import jax
import jax.numpy as jnp
from jax.experimental import pallas as pl
from jax.experimental.pallas import tpu as pltpu

Q = 256
D = 768
N_KEYS = 100000
TOPK = 5
N_BLK = 4000
N_BLOCKS = N_KEYS // N_BLK

NEG_INF = float("-inf")
BIG = 2**30


def _retrieve_kernel(q_ref, qn_ref, kb_ref, vals_out, idx_out, rv_ref, ri_ref):
    pid = pl.program_id(0)

    @pl.when(pid == 0)
    def _init():
        rv_ref[...] = jnp.full((TOPK, Q), NEG_INF, jnp.float32)
        ri_ref[...] = jnp.zeros((TOPK, Q), jnp.int32)

    kb = kb_ref[...]
    raw = jax.lax.dot_general(
        kb, q_ref[...],
        dimension_numbers=(((1,), (1,)), ((), ())),
        preferred_element_type=jnp.float32,
    )
    rkn = 1.0 / jnp.sqrt(jnp.sum(kb * kb, axis=1, keepdims=True))
    sims = raw * rkn

    base = pid * N_BLK
    row_idx = base + jax.lax.broadcasted_iota(jnp.int32, (N_BLK, Q), 0)
    big = jnp.int32(BIG)
    bvals = []
    bidxs = []
    for it in range(TOPK):
        m = jnp.max(sims, axis=0, keepdims=True)
        cand = jnp.where(sims == m, row_idx, big)
        a = jnp.min(cand, axis=0, keepdims=True)
        bvals.append(m)
        bidxs.append(a)
        if it != TOPK - 1:
            sims = jnp.where(row_idx == a, NEG_INF, sims)
    bv = jnp.concatenate(bvals, axis=0)
    bi = jnp.concatenate(bidxs, axis=0)

    cat_v = jnp.concatenate([rv_ref[...], bv], axis=0)
    cat_i = jnp.concatenate([ri_ref[...], bi], axis=0)
    mv = []
    mi = []
    for _ in range(TOPK):
        m = jnp.max(cat_v, axis=0, keepdims=True)
        a = jnp.min(jnp.where(cat_v == m, cat_i, big), axis=0, keepdims=True)
        mv.append(m)
        mi.append(a)
        cat_v = jnp.where(cat_i == a, NEG_INF, cat_v)
    rv_ref[...] = jnp.concatenate(mv, axis=0)
    ri_ref[...] = jnp.concatenate(mi, axis=0)

    @pl.when(pid == N_BLOCKS - 1)
    def _finish():
        vals_out[...] = rv_ref[...] * (1.0 / qn_ref[0:1])
        idx_out[...] = ri_ref[...]


@jax.jit
def _retrieve(queries, keys, k):
    qn = jnp.broadcast_to(
        jnp.linalg.norm(queries, axis=1)[None, :], (8, Q))
    vals_t, idx_t = pl.pallas_call(
        _retrieve_kernel,
        grid=(N_BLOCKS,),
        in_specs=[
            pl.BlockSpec((Q, D), lambda i: (0, 0)),
            pl.BlockSpec((8, Q), lambda i: (0, 0)),
            pl.BlockSpec((N_BLK, D), lambda i: (i, 0)),
        ],
        out_specs=[
            pl.BlockSpec((TOPK, Q), lambda i: (0, 0)),
            pl.BlockSpec((TOPK, Q), lambda i: (0, 0)),
        ],
        out_shape=[
            jax.ShapeDtypeStruct((TOPK, Q), jnp.float32),
            jax.ShapeDtypeStruct((TOPK, Q), jnp.int32),
        ],
        scratch_shapes=[
            pltpu.VMEM((TOPK, Q), jnp.float32),
            pltpu.VMEM((TOPK, Q), jnp.int32),
        ],
    )(queries, qn, keys)
    return vals_t.T, idx_t.T + (k - TOPK)


def kernel(queries, keys, k):
    return _retrieve(queries, keys, k)


# speedup vs baseline: 5.0389x; 1.0011x over previous
import jax
import jax.numpy as jnp
from jax.experimental import pallas as pl
from jax.experimental.pallas import tpu as pltpu

Q = 256
D = 768
N_KEYS = 100000
TOPK = 5
N_BLK = 5000
N_BLOCKS = N_KEYS // N_BLK

NEG_INF = float("-inf")
BIG = 2**30


def _retrieve_kernel(q_ref, qn_ref, kb_ref, vals_out, idx_out, rv_ref, ri_ref):
    pid = pl.program_id(0)

    @pl.when(pid == 0)
    def _init():
        rv_ref[...] = jnp.full((TOPK, Q), NEG_INF, jnp.float32)
        ri_ref[...] = jnp.zeros((TOPK, Q), jnp.int32)

    kb = kb_ref[...]
    raw = jax.lax.dot_general(
        kb, q_ref[...],
        dimension_numbers=(((1,), (1,)), ((), ())),
        preferred_element_type=jnp.float32,
    )
    rkn = 1.0 / jnp.sqrt(jnp.sum(kb * kb, axis=1, keepdims=True))
    sims = raw * rkn

    base = pid * N_BLK
    row_idx = base + jax.lax.broadcasted_iota(jnp.int32, (N_BLK, Q), 0)
    big = jnp.int32(BIG)
    bvals = []
    bidxs = []
    for it in range(TOPK):
        m = jnp.max(sims, axis=0, keepdims=True)
        cand = jnp.where(sims == m, row_idx, big)
        a = jnp.min(cand, axis=0, keepdims=True)
        bvals.append(m)
        bidxs.append(a)
        if it != TOPK - 1:
            sims = jnp.where(row_idx == a, NEG_INF, sims)
    bv = jnp.concatenate(bvals, axis=0)
    bi = jnp.concatenate(bidxs, axis=0)

    cat_v = jnp.concatenate([rv_ref[...], bv], axis=0)
    cat_i = jnp.concatenate([ri_ref[...], bi], axis=0)
    mv = []
    mi = []
    for _ in range(TOPK):
        m = jnp.max(cat_v, axis=0, keepdims=True)
        a = jnp.min(jnp.where(cat_v == m, cat_i, big), axis=0, keepdims=True)
        mv.append(m)
        mi.append(a)
        cat_v = jnp.where(cat_i == a, NEG_INF, cat_v)
    rv_ref[...] = jnp.concatenate(mv, axis=0)
    ri_ref[...] = jnp.concatenate(mi, axis=0)

    @pl.when(pid == N_BLOCKS - 1)
    def _finish():
        vals_out[...] = rv_ref[...] * (1.0 / qn_ref[0:1])
        idx_out[...] = ri_ref[...]


@jax.jit
def _retrieve(queries, keys, k):
    qn = jnp.broadcast_to(
        jnp.linalg.norm(queries, axis=1)[None, :], (8, Q))
    vals_t, idx_t = pl.pallas_call(
        _retrieve_kernel,
        grid=(N_BLOCKS,),
        in_specs=[
            pl.BlockSpec((Q, D), lambda i: (0, 0)),
            pl.BlockSpec((8, Q), lambda i: (0, 0)),
            pl.BlockSpec((N_BLK, D), lambda i: (i, 0)),
        ],
        out_specs=[
            pl.BlockSpec((TOPK, Q), lambda i: (0, 0)),
            pl.BlockSpec((TOPK, Q), lambda i: (0, 0)),
        ],
        out_shape=[
            jax.ShapeDtypeStruct((TOPK, Q), jnp.float32),
            jax.ShapeDtypeStruct((TOPK, Q), jnp.int32),
        ],
        scratch_shapes=[
            pltpu.VMEM((TOPK, Q), jnp.float32),
            pltpu.VMEM((TOPK, Q), jnp.int32),
        ],
    )(queries, qn, keys)
    return vals_t.T, idx_t.T + (k - TOPK)


def kernel(queries, keys, k):
    return _retrieve(queries, keys, k)


# speedup vs baseline: 5.0414x; 1.0005x over previous
import jax
import jax.numpy as jnp
from jax.experimental import pallas as pl
from jax.experimental.pallas import tpu as pltpu

Q = 256
D = 768
N_KEYS = 100000
TOPK = 5
N_BLK = 4000
N_BLOCKS = N_KEYS // N_BLK

NEG_INF = float("-inf")
BIG = 2**30


def _retrieve_kernel(q_ref, qn_ref, kb_ref, vals_out, idx_out, rv_ref, ri_ref):
    pid = pl.program_id(0)

    @pl.when(pid == 0)
    def _init():
        rv_ref[...] = jnp.full((TOPK, Q), NEG_INF, jnp.float32)
        ri_ref[...] = jnp.zeros((TOPK, Q), jnp.int32)

    kb = kb_ref[...]
    raw = jax.lax.dot_general(
        kb, q_ref[...],
        dimension_numbers=(((1,), (1,)), ((), ())),
        preferred_element_type=jnp.float32,
    )
    rkn = 1.0 / jnp.sqrt(jnp.sum(kb * kb, axis=1, keepdims=True))
    sims = raw * rkn

    base = pid * N_BLK
    row_idx = base + jax.lax.broadcasted_iota(jnp.int32, (N_BLK, Q), 0)
    big = jnp.int32(BIG)
    bvals = []
    bidxs = []
    for it in range(TOPK):
        m = jnp.max(sims, axis=0, keepdims=True)
        cand = jnp.where(sims == m, row_idx, big)
        a = jnp.min(cand, axis=0, keepdims=True)
        bvals.append(m)
        bidxs.append(a)
        if it != TOPK - 1:
            sims = jnp.where(row_idx == a, NEG_INF, sims)
    bv = jnp.concatenate(bvals, axis=0)
    bi = jnp.concatenate(bidxs, axis=0)

    cat_v = jnp.concatenate([rv_ref[...], bv], axis=0)
    cat_i = jnp.concatenate([ri_ref[...], bi], axis=0)
    mv = []
    mi = []
    for _ in range(TOPK):
        m = jnp.max(cat_v, axis=0, keepdims=True)
        a = jnp.min(jnp.where(cat_v == m, cat_i, big), axis=0, keepdims=True)
        mv.append(m)
        mi.append(a)
        cat_v = jnp.where(cat_i == a, NEG_INF, cat_v)
    rv_ref[...] = jnp.concatenate(mv, axis=0)
    ri_ref[...] = jnp.concatenate(mi, axis=0)

    @pl.when(pid == N_BLOCKS - 1)
    def _finish():
        vals_out[...] = rv_ref[...] * (1.0 / qn_ref[0:1])
        idx_out[...] = ri_ref[...]


@jax.jit
def _retrieve(queries, keys, k):
    qn = jnp.broadcast_to(
        jnp.linalg.norm(queries, axis=1)[None, :], (8, Q))
    vals_t, idx_t = pl.pallas_call(
        _retrieve_kernel,
        grid=(N_BLOCKS,),
        in_specs=[
            pl.BlockSpec((Q, D), lambda i: (0, 0)),
            pl.BlockSpec((8, Q), lambda i: (0, 0)),
            pl.BlockSpec((N_BLK, D), lambda i: (i, 0)),
        ],
        out_specs=[
            pl.BlockSpec((TOPK, Q), lambda i: (0, 0)),
            pl.BlockSpec((TOPK, Q), lambda i: (0, 0)),
        ],
        out_shape=[
            jax.ShapeDtypeStruct((TOPK, Q), jnp.float32),
            jax.ShapeDtypeStruct((TOPK, Q), jnp.int32),
        ],
        scratch_shapes=[
            pltpu.VMEM((TOPK, Q), jnp.float32),
            pltpu.VMEM((TOPK, Q), jnp.int32),
        ],
    )(queries, qn, keys)
    return vals_t.T, idx_t.T + (k - TOPK)


def kernel(queries, keys, k):
    return _retrieve(queries, keys, k)
